# baseline (speedup 1.0000x reference)
import functools
import jax
import jax.numpy as jnp
from jax import lax
from jax.experimental import pallas as pl
from jax.experimental.pallas import tpu as pltpu
from jax.experimental.pallas import tpu_sc as plsc

N = 10000
E = 320000
D = 128
DE = 16
H = 128
OUT = 128
L = 2

NC = 2
NS = 16
NW = NC * NS
C = 64
N_PAD = 10112
TILE_ROWS = N_PAD // NS
NCHUNK = 160
E_PER_TEC = NCHUNK * C
E_PAD = E_PER_TEC * NW
DEGW = 8
CD = 128
NCHD = E_PAD // NW // CD


def _mlp_call(x, W1, b1, W2, b2, block):
    rows = x.shape[0]
    k = x.shape[1]
    out_d = W2.shape[1]

    def body(x_ref, w1_ref, b1_ref, w2_ref, b2_ref, o_ref):
        h = jnp.maximum(
            jnp.dot(x_ref[...], w1_ref[...], preferred_element_type=jnp.float32)
            + b1_ref[...], 0.0)
        o_ref[...] = jnp.dot(h, w2_ref[...],
                             preferred_element_type=jnp.float32) + b2_ref[...]

    return pl.pallas_call(
        body,
        grid=(rows // block,),
        in_specs=[
            pl.BlockSpec((block, k), lambda i: (i, 0)),
            pl.BlockSpec(W1.shape, lambda i: (0, 0)),
            pl.BlockSpec((1, W1.shape[1]), lambda i: (0, 0)),
            pl.BlockSpec(W2.shape, lambda i: (0, 0)),
            pl.BlockSpec((1, out_d), lambda i: (0, 0)),
        ],
        out_specs=pl.BlockSpec((block, out_d), lambda i: (i, 0)),
        out_shape=jax.ShapeDtypeStruct((rows, out_d), jnp.float32),
    )(x, W1, b1[None], W2, b2[None])


def _ab_call(h, W1i, W1j, b1, block=1264):
    rows = h.shape[0]

    def body(h_ref, wi_ref, wj_ref, b1_ref, a_ref, b_ref):
        hb = h_ref[...]
        a_ref[...] = jnp.dot(hb, wi_ref[...],
                             preferred_element_type=jnp.float32) + b1_ref[...]
        b_ref[...] = jnp.dot(hb, wj_ref[...],
                             preferred_element_type=jnp.float32)

    return pl.pallas_call(
        body,
        grid=(rows // block,),
        in_specs=[
            pl.BlockSpec((block, H), lambda i: (i, 0)),
            pl.BlockSpec((H, H), lambda i: (0, 0)),
            pl.BlockSpec((H, H), lambda i: (0, 0)),
            pl.BlockSpec((1, H), lambda i: (0, 0)),
        ],
        out_specs=[
            pl.BlockSpec((block, H), lambda i: (i, 0)),
            pl.BlockSpec((block, H), lambda i: (i, 0)),
        ],
        out_shape=[
            jax.ShapeDtypeStruct((rows, H), jnp.float32),
            jax.ShapeDtypeStruct((rows, H), jnp.float32),
        ],
    )(h, W1i, W1j, b1[None])


def _ce_call(ea, W1e, block=2048):
    rows = ea.shape[0]

    def body(e_ref, w_ref, o_ref):
        o_ref[...] = jnp.dot(e_ref[...], w_ref[...],
                             preferred_element_type=jnp.float32)

    return pl.pallas_call(
        body,
        grid=(rows // block,),
        in_specs=[
            pl.BlockSpec((block, DE), lambda i: (i, 0)),
            pl.BlockSpec((DE, H), lambda i: (0, 0)),
        ],
        out_specs=pl.BlockSpec((block, H), lambda i: (i, 0)),
        out_shape=jax.ShapeDtypeStruct((rows, H), jnp.float32),
    )(ea, W1e)


def _update_call(Sp, degt, h, W2, b2, U1t, U1b, ub1, U2, ub2, block=1264):
    rows = h.shape[0]

    def body(sp_ref, dg_ref, h_ref, w2_ref, b2_ref, u1t_ref, u1b_ref,
             ub1_ref, u2_ref, ub2_ref, o_ref):
        S = sp_ref[0] + sp_ref[1]
        deg = dg_ref[0, :, :1] + dg_ref[1, :, :1]
        agg = jnp.dot(S, w2_ref[...],
                      preferred_element_type=jnp.float32) + deg * b2_ref[...]
        t = jnp.maximum(
            jnp.dot(h_ref[...], u1t_ref[...], preferred_element_type=jnp.float32)
            + jnp.dot(agg, u1b_ref[...], preferred_element_type=jnp.float32)
            + ub1_ref[...], 0.0)
        o_ref[...] = jnp.dot(t, u2_ref[...],
                             preferred_element_type=jnp.float32) + ub2_ref[...]

    full = lambda shape: pl.BlockSpec(shape, lambda i: tuple(0 for _ in shape))
    return pl.pallas_call(
        body,
        grid=(rows // block,),
        in_specs=[
            pl.BlockSpec((NC, block, H), lambda i: (0, i, 0)),
            pl.BlockSpec((NC, block, DEGW), lambda i: (0, i, 0)),
            pl.BlockSpec((block, H), lambda i: (i, 0)),
            full((H, H)), full((1, H)), full((H, H)), full((H, H)),
            full((1, H)), full((H, H)), full((1, H)),
        ],
        out_specs=pl.BlockSpec((block, H), lambda i: (i, 0)),
        out_shape=jax.ShapeDtypeStruct((rows, H), jnp.float32),
    )(Sp, degt, h, W2, b2[None], U1t, U1b, ub1[None], U2, ub2[None])


def _make_sc_kernel():
    mesh = plsc.VectorSubcoreMesh(core_axis_name="c", subcore_axis_name="s")

    def body(A_hbm, B_hbm, Ce_hbm, dst_hbm, src_hbm, z_hbm,
             S_out, d0, d1, s0, s1, bufA, bufB, bufC, S_sh,
             sem_i0, sem_i1, sem_i2, sem_i3,
             sem_a0, sem_b0, sem_c0, sem_a1, sem_b1, sem_c1,
             sem_s0, sem_s1):
        c = lax.axis_index("c")
        s = lax.axis_index("s")
        wid = s * NC + c
        row_base = s * TILE_ROWS
        ebase0 = wid * E_PER_TEC
        dI = (d0, d1)
        sI = (s0, s1)
        bA = (bufA.at[0], bufA.at[1])
        bB = (bufB.at[0], bufB.at[1])
        bC = (bufC.at[0], bufC.at[1])
        sem_g = ((sem_a0, sem_b0, sem_c0), (sem_a1, sem_b1, sem_c1))
        sem_s = (sem_s0, sem_s1)

        def compute(slot):
            a, b, cc = bA[slot], bB[slot], bC[slot]

            def row_body(r, carry):
                for rr in range(2):
                    row = 2 * r + rr
                    for j in range(H // 16):
                        sl = pl.ds(j * 16, 16)
                        cc[row, sl] = jnp.maximum(
                            a[row, sl] + b[row, sl] + cc[row, sl], 0.0)
                return carry

            lax.fori_loop(0, C // 2, row_body, 0)

        pltpu.sync_copy(z_hbm, S_sh.at[pl.ds(row_base, TILE_ROWS)])
        plsc.subcore_barrier()

        def pair_body(p, carry):
            e0 = pl.ds(ebase0 + (2 * p) * C, C)
            e1 = pl.ds(ebase0 + (2 * p + 1) * C, C)
            ix = (pltpu.async_copy(dst_hbm.at[e0], d0, sem_i0),
                  pltpu.async_copy(src_hbm.at[e0], s0, sem_i1),
                  pltpu.async_copy(dst_hbm.at[e1], d1, sem_i2),
                  pltpu.async_copy(src_hbm.at[e1], s1, sem_i3))
            for d in ix:
                d.wait()
            g0 = (pltpu.async_copy(A_hbm.at[d0], bA[0], sem_a0),
                  pltpu.async_copy(B_hbm.at[s0], bB[0], sem_b0),
                  pltpu.async_copy(Ce_hbm.at[e0], bC[0], sem_c0))
            g1 = (pltpu.async_copy(A_hbm.at[d1], bA[1], sem_a1),
                  pltpu.async_copy(B_hbm.at[s1], bB[1], sem_b1),
                  pltpu.async_copy(Ce_hbm.at[e1], bC[1], sem_c1))
            for d in g0:
                d.wait()
            compute(0)
            sc0 = pltpu.async_copy(bC[0], S_sh.at[d0], sem_s0, add=True)
            for d in g1:
                d.wait()
            compute(1)
            sc1 = pltpu.async_copy(bC[1], S_sh.at[d1], sem_s1, add=True)
            sc0.wait()
            sc1.wait()
            return carry

        lax.fori_loop(0, NCHUNK // 2, pair_body, 0)
        plsc.subcore_barrier()

        sl = pl.ds(row_base, TILE_ROWS)
        pltpu.sync_copy(S_sh.at[sl], S_out.at[c, sl])

    return pl.kernel(
        body,
        out_type=jax.ShapeDtypeStruct((NC, N_PAD, H), jnp.float32),
        mesh=mesh,
        scratch_types=(
            pltpu.VMEM((C,), jnp.int32),
            pltpu.VMEM((C,), jnp.int32),
            pltpu.VMEM((C,), jnp.int32),
            pltpu.VMEM((C,), jnp.int32),
            pltpu.VMEM((2, C, H), jnp.float32),
            pltpu.VMEM((2, C, H), jnp.float32),
            pltpu.VMEM((2, C, H), jnp.float32),
            pltpu.VMEM_SHARED((N_PAD, H), jnp.float32),
        ) + (pltpu.SemaphoreType.DMA,) * 12)


def _make_deg_kernel():
    mesh = plsc.VectorSubcoreMesh(core_axis_name="c", subcore_axis_name="s")

    def body(dst_hbm, z8_hbm, ones_hbm,
             deg_out, dst_v, ones_v, deg_sh):
        c = lax.axis_index("c")
        s = lax.axis_index("s")
        wid = s * NC + c
        row_base = s * TILE_ROWS

        pltpu.sync_copy(z8_hbm, deg_sh.at[pl.ds(row_base, TILE_ROWS)])
        pltpu.sync_copy(ones_hbm, ones_v)
        plsc.subcore_barrier()

        def chunk_body(i, carry):
            ebase = wid * (NCHD * CD) + i * CD
            pltpu.sync_copy(dst_hbm.at[pl.ds(ebase, CD)], dst_v)
            pltpu.sync_copy(ones_v, deg_sh.at[dst_v], add=True)
            return carry

        lax.fori_loop(0, NCHD, chunk_body, 0)
        plsc.subcore_barrier()

        sl = pl.ds(row_base, TILE_ROWS)
        pltpu.sync_copy(deg_sh.at[sl], deg_out.at[c, sl])

    return pl.kernel(
        body,
        out_type=jax.ShapeDtypeStruct((NC, N_PAD, DEGW), jnp.float32),
        mesh=mesh,
        scratch_types=(
            pltpu.VMEM((CD,), jnp.int32),
            pltpu.VMEM((CD, DEGW), jnp.float32),
            pltpu.VMEM_SHARED((N_PAD, DEGW), jnp.float32),
        ))


_sc_edge = _make_sc_kernel()
_sc_deg = _make_deg_kernel()


def kernel(x, edge_index, batch, edge_attr, params):
    p = params
    src = edge_index[0]
    dst = edge_index[1]
    dst_p = jnp.concatenate(
        [dst, jnp.full((E_PAD - E,), N, dtype=jnp.int32)])
    src_p = jnp.concatenate(
        [src, jnp.zeros((E_PAD - E,), dtype=jnp.int32)])

    ea_p = jnp.pad(edge_attr, ((0, E_PAD - E), (0, 0)))
    x_p = jnp.pad(x, ((0, N_PAD - N), (0, 0)))

    zeros = jnp.zeros((TILE_ROWS, H), jnp.float32)
    zeros8 = jnp.zeros((TILE_ROWS, DEGW), jnp.float32)
    ones = jnp.ones((CD, DEGW), jnp.float32)

    h = _mlp_call(x_p, p['emb_W1'], p['emb_b1'], p['emb_W2'], p['emb_b2'],
                  block=1264)

    degt = _sc_deg(dst_p, zeros8, ones)
    for l in range(L):
        W1 = p[f'l{l}_msg_W1']
        A, B = _ab_call(h, W1[:H], W1[H:2 * H], p[f'l{l}_msg_b1'])
        Ce = _ce_call(ea_p, W1[2 * H:])
        Sp = _sc_edge(A, B, Ce, dst_p, src_p, zeros)
        U1 = p[f'l{l}_upd_W1']
        h = _update_call(Sp, degt, h, p[f'l{l}_msg_W2'], p[f'l{l}_msg_b2'],
                         U1[:H], U1[H:], p[f'l{l}_upd_b1'],
                         p[f'l{l}_upd_W2'], p[f'l{l}_upd_b2'])

    out = _mlp_call(h, p['head_W1'], p['head_b1'], p['head_W2'],
                    p['head_b2'], block=1264)
    return out[:N]


# speedup vs baseline: 1.2081x; 1.2081x over previous
import functools
import jax
import jax.numpy as jnp
from jax import lax
from jax.experimental import pallas as pl
from jax.experimental.pallas import tpu as pltpu
from jax.experimental.pallas import tpu_sc as plsc

N = 10000
E = 320000
D = 128
DE = 16
H = 128
OUT = 128
L = 2

NC = 2
NS = 16
NW = NC * NS
C = 64
N_PAD = 10112
TILE_ROWS = N_PAD // NS
NCHUNK = 160
E_PER_TEC = NCHUNK * C
E_PAD = E_PER_TEC * NW
DEGW = 8
CD = 128
NCHD = E_PAD // NW // CD


def _mlp_call(x, W1, b1, W2, b2, block):
    rows = x.shape[0]
    k = x.shape[1]
    out_d = W2.shape[1]

    def body(x_ref, w1_ref, b1_ref, w2_ref, b2_ref, o_ref):
        h = jnp.maximum(
            jnp.dot(x_ref[...], w1_ref[...], preferred_element_type=jnp.float32)
            + b1_ref[...], 0.0)
        o_ref[...] = jnp.dot(h, w2_ref[...],
                             preferred_element_type=jnp.float32) + b2_ref[...]

    return pl.pallas_call(
        body,
        grid=(rows // block,),
        in_specs=[
            pl.BlockSpec((block, k), lambda i: (i, 0)),
            pl.BlockSpec(W1.shape, lambda i: (0, 0)),
            pl.BlockSpec((1, W1.shape[1]), lambda i: (0, 0)),
            pl.BlockSpec(W2.shape, lambda i: (0, 0)),
            pl.BlockSpec((1, out_d), lambda i: (0, 0)),
        ],
        out_specs=pl.BlockSpec((block, out_d), lambda i: (i, 0)),
        out_shape=jax.ShapeDtypeStruct((rows, out_d), jnp.float32),
    )(x, W1, b1[None], W2, b2[None])


def _ab_call(h, W1i, W1j, b1, block=1264):
    rows = h.shape[0]

    def body(h_ref, wi_ref, wj_ref, b1_ref, a_ref, b_ref):
        hb = h_ref[...]
        a_ref[...] = jnp.dot(hb, wi_ref[...],
                             preferred_element_type=jnp.float32) + b1_ref[...]
        b_ref[...] = jnp.dot(hb, wj_ref[...],
                             preferred_element_type=jnp.float32)

    return pl.pallas_call(
        body,
        grid=(rows // block,),
        in_specs=[
            pl.BlockSpec((block, H), lambda i: (i, 0)),
            pl.BlockSpec((H, H), lambda i: (0, 0)),
            pl.BlockSpec((H, H), lambda i: (0, 0)),
            pl.BlockSpec((1, H), lambda i: (0, 0)),
        ],
        out_specs=[
            pl.BlockSpec((block, H), lambda i: (i, 0)),
            pl.BlockSpec((block, H), lambda i: (i, 0)),
        ],
        out_shape=[
            jax.ShapeDtypeStruct((rows, H), jnp.float32),
            jax.ShapeDtypeStruct((rows, H), jnp.float32),
        ],
    )(h, W1i, W1j, b1[None])


def _ce_call(ea, W1e, block=2048):
    rows = ea.shape[0]

    def body(e_ref, w_ref, o_ref):
        o_ref[...] = jnp.dot(e_ref[...], w_ref[...],
                             preferred_element_type=jnp.float32)

    return pl.pallas_call(
        body,
        grid=(rows // block,),
        in_specs=[
            pl.BlockSpec((block, DE), lambda i: (i, 0)),
            pl.BlockSpec((DE, H), lambda i: (0, 0)),
        ],
        out_specs=pl.BlockSpec((block, H), lambda i: (i, 0)),
        out_shape=jax.ShapeDtypeStruct((rows, H), jnp.float32),
    )(ea, W1e)


def _update_call(Sp, degt, h, W2, b2, U1t, U1b, ub1, U2, ub2, block=1264):
    rows = h.shape[0]

    def body(sp_ref, dg_ref, h_ref, w2_ref, b2_ref, u1t_ref, u1b_ref,
             ub1_ref, u2_ref, ub2_ref, o_ref):
        S = sp_ref[0] + sp_ref[1]
        deg = dg_ref[0, :, :1] + dg_ref[1, :, :1]
        agg = jnp.dot(S, w2_ref[...],
                      preferred_element_type=jnp.float32) + deg * b2_ref[...]
        t = jnp.maximum(
            jnp.dot(h_ref[...], u1t_ref[...], preferred_element_type=jnp.float32)
            + jnp.dot(agg, u1b_ref[...], preferred_element_type=jnp.float32)
            + ub1_ref[...], 0.0)
        o_ref[...] = jnp.dot(t, u2_ref[...],
                             preferred_element_type=jnp.float32) + ub2_ref[...]

    full = lambda shape: pl.BlockSpec(shape, lambda i: tuple(0 for _ in shape))
    return pl.pallas_call(
        body,
        grid=(rows // block,),
        in_specs=[
            pl.BlockSpec((NC, block, H), lambda i: (0, i, 0)),
            pl.BlockSpec((NC, block, DEGW), lambda i: (0, i, 0)),
            pl.BlockSpec((block, H), lambda i: (i, 0)),
            full((H, H)), full((1, H)), full((H, H)), full((H, H)),
            full((1, H)), full((H, H)), full((1, H)),
        ],
        out_specs=pl.BlockSpec((block, H), lambda i: (i, 0)),
        out_shape=jax.ShapeDtypeStruct((rows, H), jnp.float32),
    )(Sp, degt, h, W2, b2[None], U1t, U1b, ub1[None], U2, ub2[None])


def _make_sc_kernel():
    mesh = plsc.VectorSubcoreMesh(core_axis_name="c", subcore_axis_name="s")

    def body(A_hbm, B_hbm, Ce_hbm, dst_hbm, src_hbm, z_hbm,
             S_out, dst_r, src_r, bufA, bufB, bufC, S_sh, *sems):
        c = lax.axis_index("c")
        s = lax.axis_index("s")
        wid = s * NC + c
        row_base = s * TILE_ROWS
        ebase0 = wid * E_PER_TEC
        sem_id = sems[0:4]
        sem_is = sems[4:8]
        sem_g = (sems[8:11], sems[11:14])
        sem_s = sems[14:16]
        dI = tuple(dst_r.at[q] for q in range(4))
        sI = tuple(src_r.at[q] for q in range(4))
        bA = (bufA.at[0], bufA.at[1])
        bB = (bufB.at[0], bufB.at[1])
        bC = (bufC.at[0], bufC.at[1])
        idummy = dst_hbm.at[pl.ds(0, C)]
        gdummy = Ce_hbm.at[pl.ds(0, C)]

        def issue_idx(k, q):
            esl = pl.ds(ebase0 + k * C, C)
            pltpu.async_copy(dst_hbm.at[esl], dI[q], sem_id[q])
            pltpu.async_copy(src_hbm.at[esl], sI[q], sem_is[q])

        def wait_idx(q):
            pltpu.make_async_copy(idummy, dI[q], sem_id[q]).wait()
            pltpu.make_async_copy(idummy, sI[q], sem_is[q]).wait()

        def issue_gathers(k, slot, q):
            pltpu.async_copy(A_hbm.at[dI[q]], bA[slot], sem_g[slot][0])
            pltpu.async_copy(B_hbm.at[sI[q]], bB[slot], sem_g[slot][1])
            pltpu.async_copy(Ce_hbm.at[pl.ds(ebase0 + k * C, C)],
                             bC[slot], sem_g[slot][2])

        def wait_gathers(slot):
            for i, buf in enumerate((bA[slot], bB[slot], bC[slot])):
                pltpu.make_async_copy(gdummy, buf, sem_g[slot][i]).wait()

        def wait_scatter(slot):
            pltpu.make_async_copy(gdummy, bC[slot], sem_s[slot]).wait()

        def compute(slot):
            a, b, cc = bA[slot], bB[slot], bC[slot]

            def row_body(r, carry):
                for rr in range(2):
                    row = 2 * r + rr
                    for j in range(H // 16):
                        sl = pl.ds(j * 16, 16)
                        cc[row, sl] = jnp.maximum(
                            a[row, sl] + b[row, sl] + cc[row, sl], 0.0)
                return carry

            lax.fori_loop(0, C // 2, row_body, 0)

        pltpu.sync_copy(z_hbm, S_sh.at[pl.ds(row_base, TILE_ROWS)])
        i0d = pltpu.async_copy(dst_hbm.at[pl.ds(ebase0, C)], dI[0],
                               sem_id[0])
        i0s = pltpu.async_copy(src_hbm.at[pl.ds(ebase0, C)], sI[0],
                               sem_is[0])
        issue_idx(1, 1)
        i0d.wait()
        i0s.wait()
        issue_gathers(0, 0, 0)
        plsc.subcore_barrier()

        NG = NCHUNK // 4

        def quad_body(g, carry):
            for u in range(4):
                k = 4 * g + u
                slot = u % 2

                if u < 2:
                    @pl.when(g > 0)
                    def _():
                        wait_scatter(slot)
                else:
                    wait_scatter(slot)

                if u < 2:
                    issue_idx(k + 2, (u + 2) % 4)
                else:
                    @pl.when(g < NG - 1)
                    def _():
                        issue_idx(k + 2, (u + 2) % 4)

                if u < 3:
                    wait_idx((u + 1) % 4)
                    issue_gathers(k + 1, 1 - slot, (u + 1) % 4)
                else:
                    @pl.when(g < NG - 1)
                    def _():
                        wait_idx((u + 1) % 4)
                        issue_gathers(k + 1, 1 - slot, (u + 1) % 4)

                wait_gathers(slot)
                compute(slot)
                pltpu.async_copy(bC[slot], S_sh.at[dI[u]], sem_s[slot],
                                 add=True)
            return carry

        lax.fori_loop(0, NG, quad_body, 0)
        wait_scatter(0)
        wait_scatter(1)
        plsc.subcore_barrier()

        sl = pl.ds(row_base, TILE_ROWS)
        pltpu.sync_copy(S_sh.at[sl], S_out.at[c, sl])

    return pl.kernel(
        body,
        out_type=jax.ShapeDtypeStruct((NC, N_PAD, H), jnp.float32),
        mesh=mesh,
        scratch_types=(
            pltpu.VMEM((4, C), jnp.int32),
            pltpu.VMEM((4, C), jnp.int32),
            pltpu.VMEM((2, C, H), jnp.float32),
            pltpu.VMEM((2, C, H), jnp.float32),
            pltpu.VMEM((2, C, H), jnp.float32),
            pltpu.VMEM_SHARED((N_PAD, H), jnp.float32),
        ) + (pltpu.SemaphoreType.DMA,) * 16)


def _make_deg_kernel():
    mesh = plsc.VectorSubcoreMesh(core_axis_name="c", subcore_axis_name="s")

    def body(dst_hbm, z8_hbm, ones_hbm,
             deg_out, dst_v, ones_v, deg_sh):
        c = lax.axis_index("c")
        s = lax.axis_index("s")
        wid = s * NC + c
        row_base = s * TILE_ROWS

        pltpu.sync_copy(z8_hbm, deg_sh.at[pl.ds(row_base, TILE_ROWS)])
        pltpu.sync_copy(ones_hbm, ones_v)
        plsc.subcore_barrier()

        def chunk_body(i, carry):
            ebase = wid * (NCHD * CD) + i * CD
            pltpu.sync_copy(dst_hbm.at[pl.ds(ebase, CD)], dst_v)
            pltpu.sync_copy(ones_v, deg_sh.at[dst_v], add=True)
            return carry

        lax.fori_loop(0, NCHD, chunk_body, 0)
        plsc.subcore_barrier()

        sl = pl.ds(row_base, TILE_ROWS)
        pltpu.sync_copy(deg_sh.at[sl], deg_out.at[c, sl])

    return pl.kernel(
        body,
        out_type=jax.ShapeDtypeStruct((NC, N_PAD, DEGW), jnp.float32),
        mesh=mesh,
        scratch_types=(
            pltpu.VMEM((CD,), jnp.int32),
            pltpu.VMEM((CD, DEGW), jnp.float32),
            pltpu.VMEM_SHARED((N_PAD, DEGW), jnp.float32),
        ))


_sc_edge = _make_sc_kernel()
_sc_deg = _make_deg_kernel()


def kernel(x, edge_index, batch, edge_attr, params):
    p = params
    src = edge_index[0]
    dst = edge_index[1]
    dst_p = jnp.concatenate(
        [dst, jnp.full((E_PAD - E,), N, dtype=jnp.int32)])
    src_p = jnp.concatenate(
        [src, jnp.zeros((E_PAD - E,), dtype=jnp.int32)])

    ea_p = jnp.pad(edge_attr, ((0, E_PAD - E), (0, 0)))
    x_p = jnp.pad(x, ((0, N_PAD - N), (0, 0)))

    zeros = jnp.zeros((TILE_ROWS, H), jnp.float32)
    zeros8 = jnp.zeros((TILE_ROWS, DEGW), jnp.float32)
    ones = jnp.ones((CD, DEGW), jnp.float32)

    h = _mlp_call(x_p, p['emb_W1'], p['emb_b1'], p['emb_W2'], p['emb_b2'],
                  block=1264)

    degt = _sc_deg(dst_p, zeros8, ones)
    for l in range(L):
        W1 = p[f'l{l}_msg_W1']
        A, B = _ab_call(h, W1[:H], W1[H:2 * H], p[f'l{l}_msg_b1'])
        Ce = _ce_call(ea_p, W1[2 * H:])
        Sp = _sc_edge(A, B, Ce, dst_p, src_p, zeros)
        U1 = p[f'l{l}_upd_W1']
        h = _update_call(Sp, degt, h, p[f'l{l}_msg_W2'], p[f'l{l}_msg_b2'],
                         U1[:H], U1[H:], p[f'l{l}_upd_b1'],
                         p[f'l{l}_upd_W2'], p[f'l{l}_upd_b2'])

    out = _mlp_call(h, p['head_W1'], p['head_b1'], p['head_W2'],
                    p['head_b2'], block=1264)
    return out[:N]


# speedup vs baseline: 1.2094x; 1.0011x over previous
import functools
import jax
import jax.numpy as jnp
from jax import lax
from jax.experimental import pallas as pl
from jax.experimental.pallas import tpu as pltpu
from jax.experimental.pallas import tpu_sc as plsc

N = 10000
E = 320000
D = 128
DE = 16
H = 128
OUT = 128
L = 2

NC = 2
NS = 16
NW = NC * NS
C = 64
N_PAD = 10112
TILE_ROWS = N_PAD // NS
NCHUNK = 160
E_PER_TEC = NCHUNK * C
E_PAD = E_PER_TEC * NW
DEGW = 8
CD = 128
NCHD = E_PAD // NW // CD


def _mlp_call(x, W1, b1, W2, b2, block):
    rows = x.shape[0]
    k = x.shape[1]
    out_d = W2.shape[1]

    def body(x_ref, w1_ref, b1_ref, w2_ref, b2_ref, o_ref):
        h = jnp.maximum(
            jnp.dot(x_ref[...], w1_ref[...], preferred_element_type=jnp.float32)
            + b1_ref[...], 0.0)
        o_ref[...] = jnp.dot(h, w2_ref[...],
                             preferred_element_type=jnp.float32) + b2_ref[...]

    return pl.pallas_call(
        body,
        grid=(rows // block,),
        in_specs=[
            pl.BlockSpec((block, k), lambda i: (i, 0)),
            pl.BlockSpec(W1.shape, lambda i: (0, 0)),
            pl.BlockSpec((1, W1.shape[1]), lambda i: (0, 0)),
            pl.BlockSpec(W2.shape, lambda i: (0, 0)),
            pl.BlockSpec((1, out_d), lambda i: (0, 0)),
        ],
        out_specs=pl.BlockSpec((block, out_d), lambda i: (i, 0)),
        out_shape=jax.ShapeDtypeStruct((rows, out_d), jnp.float32),
    )(x, W1, b1[None], W2, b2[None])


def _ab_call(h, W1i, W1j, b1, block=1264):
    rows = h.shape[0]

    def body(h_ref, wi_ref, wj_ref, b1_ref, a_ref, b_ref):
        hb = h_ref[...]
        a_ref[...] = jnp.dot(hb, wi_ref[...],
                             preferred_element_type=jnp.float32) + b1_ref[...]
        b_ref[...] = jnp.dot(hb, wj_ref[...],
                             preferred_element_type=jnp.float32)

    return pl.pallas_call(
        body,
        grid=(rows // block,),
        in_specs=[
            pl.BlockSpec((block, H), lambda i: (i, 0)),
            pl.BlockSpec((H, H), lambda i: (0, 0)),
            pl.BlockSpec((H, H), lambda i: (0, 0)),
            pl.BlockSpec((1, H), lambda i: (0, 0)),
        ],
        out_specs=[
            pl.BlockSpec((block, H), lambda i: (i, 0)),
            pl.BlockSpec((block, H), lambda i: (i, 0)),
        ],
        out_shape=[
            jax.ShapeDtypeStruct((rows, H), jnp.float32),
            jax.ShapeDtypeStruct((rows, H), jnp.float32),
        ],
    )(h, W1i, W1j, b1[None])


def _ce_call(ea, W1e, block=2048):
    rows = ea.shape[0]

    def body(e_ref, w_ref, o_ref):
        o_ref[...] = jnp.dot(e_ref[...], w_ref[...],
                             preferred_element_type=jnp.float32)

    return pl.pallas_call(
        body,
        grid=(rows // block,),
        in_specs=[
            pl.BlockSpec((block, DE), lambda i: (i, 0)),
            pl.BlockSpec((DE, H), lambda i: (0, 0)),
        ],
        out_specs=pl.BlockSpec((block, H), lambda i: (i, 0)),
        out_shape=jax.ShapeDtypeStruct((rows, H), jnp.float32),
    )(ea, W1e)


def _update_call(Sp, degt, h, W2, b2, U1t, U1b, ub1, U2, ub2, block=1264):
    rows = h.shape[0]

    def body(sp_ref, dg_ref, h_ref, w2_ref, b2_ref, u1t_ref, u1b_ref,
             ub1_ref, u2_ref, ub2_ref, o_ref):
        S = sp_ref[0] + sp_ref[1]
        deg = dg_ref[0, :, :1] + dg_ref[1, :, :1]
        agg = jnp.dot(S, w2_ref[...],
                      preferred_element_type=jnp.float32) + deg * b2_ref[...]
        t = jnp.maximum(
            jnp.dot(h_ref[...], u1t_ref[...], preferred_element_type=jnp.float32)
            + jnp.dot(agg, u1b_ref[...], preferred_element_type=jnp.float32)
            + ub1_ref[...], 0.0)
        o_ref[...] = jnp.dot(t, u2_ref[...],
                             preferred_element_type=jnp.float32) + ub2_ref[...]

    full = lambda shape: pl.BlockSpec(shape, lambda i: tuple(0 for _ in shape))
    return pl.pallas_call(
        body,
        grid=(rows // block,),
        in_specs=[
            pl.BlockSpec((NC, block, H), lambda i: (0, i, 0)),
            pl.BlockSpec((NC, block, DEGW), lambda i: (0, i, 0)),
            pl.BlockSpec((block, H), lambda i: (i, 0)),
            full((H, H)), full((1, H)), full((H, H)), full((H, H)),
            full((1, H)), full((H, H)), full((1, H)),
        ],
        out_specs=pl.BlockSpec((block, H), lambda i: (i, 0)),
        out_shape=jax.ShapeDtypeStruct((rows, H), jnp.float32),
    )(Sp, degt, h, W2, b2[None], U1t, U1b, ub1[None], U2, ub2[None])


def _make_sc_kernel():
    mesh = plsc.VectorSubcoreMesh(core_axis_name="c", subcore_axis_name="s")

    def body(A_hbm, B_hbm, Ce_hbm, dst_hbm, src_hbm, z_hbm,
             S_out, dst_r, src_r, bufA, bufB, bufC, S_sh, *sems):
        c = lax.axis_index("c")
        s = lax.axis_index("s")
        wid = s * NC + c
        row_base = s * TILE_ROWS
        ebase0 = wid * E_PER_TEC
        sem_id = sems[0:4]
        sem_is = sems[4:8]
        sem_g = (sems[8:11], sems[11:14])
        sem_s = sems[14:16]
        dI = tuple(dst_r.at[q] for q in range(4))
        sI = tuple(src_r.at[q] for q in range(4))
        bA = (bufA.at[0], bufA.at[1])
        bB = (bufB.at[0], bufB.at[1])
        bC = (bufC.at[0], bufC.at[1])
        idummy = dst_hbm.at[pl.ds(0, C)]
        gdummy = Ce_hbm.at[pl.ds(0, C)]

        def issue_idx(k, q):
            esl = pl.ds(ebase0 + k * C, C)
            pltpu.async_copy(dst_hbm.at[esl], dI[q], sem_id[q])
            pltpu.async_copy(src_hbm.at[esl], sI[q], sem_is[q])

        def wait_idx(q):
            pltpu.make_async_copy(idummy, dI[q], sem_id[q]).wait()
            pltpu.make_async_copy(idummy, sI[q], sem_is[q]).wait()

        def issue_gathers(k, slot, q):
            pltpu.async_copy(A_hbm.at[dI[q]], bA[slot], sem_g[slot][0])
            pltpu.async_copy(B_hbm.at[sI[q]], bB[slot], sem_g[slot][1])
            pltpu.async_copy(Ce_hbm.at[pl.ds(ebase0 + k * C, C)],
                             bC[slot], sem_g[slot][2])

        def wait_gathers(slot):
            for i, buf in enumerate((bA[slot], bB[slot], bC[slot])):
                pltpu.make_async_copy(gdummy, buf, sem_g[slot][i]).wait()

        def wait_scatter(slot):
            pltpu.make_async_copy(gdummy, bC[slot], sem_s[slot]).wait()

        def compute(slot):
            a, b, cc = bA[slot], bB[slot], bC[slot]

            def row_body(r, carry):
                for rr in range(2):
                    row = 2 * r + rr
                    for j in range(H // 16):
                        sl = pl.ds(j * 16, 16)
                        cc[row, sl] = jnp.maximum(
                            a[row, sl] + b[row, sl] + cc[row, sl], 0.0)
                return carry

            lax.fori_loop(0, C // 2, row_body, 0)

        pltpu.sync_copy(z_hbm, S_sh.at[pl.ds(row_base, TILE_ROWS)])
        i0d = pltpu.async_copy(dst_hbm.at[pl.ds(ebase0, C)], dI[0],
                               sem_id[0])
        i0s = pltpu.async_copy(src_hbm.at[pl.ds(ebase0, C)], sI[0],
                               sem_is[0])
        issue_idx(1, 1)
        i0d.wait()
        i0s.wait()
        issue_gathers(0, 0, 0)
        plsc.subcore_barrier()

        NG = NCHUNK // 4

        def quad_body(g, carry):
            for u in range(4):
                k = 4 * g + u
                slot = u % 2

                if u < 2:
                    @pl.when(g > 0)
                    def _():
                        wait_scatter(slot)
                else:
                    wait_scatter(slot)

                if u < 2:
                    issue_idx(k + 2, (u + 2) % 4)
                else:
                    @pl.when(g < NG - 1)
                    def _():
                        issue_idx(k + 2, (u + 2) % 4)

                if u < 3:
                    wait_idx((u + 1) % 4)
                    issue_gathers(k + 1, 1 - slot, (u + 1) % 4)
                else:
                    @pl.when(g < NG - 1)
                    def _():
                        wait_idx((u + 1) % 4)
                        issue_gathers(k + 1, 1 - slot, (u + 1) % 4)

                wait_gathers(slot)
                compute(slot)
                pltpu.async_copy(bC[slot], S_sh.at[dI[u]], sem_s[slot],
                                 add=True)
            return carry

        lax.fori_loop(0, NG, quad_body, 0)
        wait_scatter(0)
        wait_scatter(1)
        plsc.subcore_barrier()

        sl = pl.ds(row_base, TILE_ROWS)
        pltpu.sync_copy(S_sh.at[sl], S_out.at[c, sl])

    return pl.kernel(
        body,
        out_type=jax.ShapeDtypeStruct((NC, N_PAD, H), jnp.float32),
        mesh=mesh,
        scratch_types=(
            pltpu.VMEM((4, C), jnp.int32),
            pltpu.VMEM((4, C), jnp.int32),
            pltpu.VMEM((2, C, H), jnp.float32),
            pltpu.VMEM((2, C, H), jnp.float32),
            pltpu.VMEM((2, C, H), jnp.float32),
            pltpu.VMEM_SHARED((N_PAD, H), jnp.float32),
        ) + (pltpu.SemaphoreType.DMA,) * 16)


def _make_deg_kernel():
    mesh = plsc.VectorSubcoreMesh(core_axis_name="c", subcore_axis_name="s")

    def body(dst_hbm, z8_hbm, ones_hbm,
             deg_out, dst_v, ones_v, deg_sh, sem_i0, sem_i1):
        sem_i = (sem_i0, sem_i1)
        c = lax.axis_index("c")
        s = lax.axis_index("s")
        wid = s * NC + c
        row_base = s * TILE_ROWS

        ebase0 = wid * (NCHD * CD)
        dI = (dst_v.at[0], dst_v.at[1])
        idummy = dst_hbm.at[pl.ds(0, CD)]

        def issue_idx(k, q):
            pltpu.async_copy(dst_hbm.at[pl.ds(ebase0 + k * CD, CD)],
                             dI[q], sem_i[q])

        pltpu.sync_copy(z8_hbm, deg_sh.at[pl.ds(row_base, TILE_ROWS)])
        pltpu.sync_copy(ones_hbm, ones_v)
        issue_idx(0, 0)
        plsc.subcore_barrier()

        def pair_body(g, carry):
            for u in range(2):
                k = 2 * g + u
                if u == 0:
                    issue_idx(k + 1, 1)
                else:
                    @pl.when(g < NCHD // 2 - 1)
                    def _():
                        issue_idx(k + 1, 0)
                pltpu.make_async_copy(idummy, dI[u], sem_i[u]).wait()
                pltpu.sync_copy(ones_v, deg_sh.at[dI[u]], add=True)
            return carry

        lax.fori_loop(0, NCHD // 2, pair_body, 0)
        plsc.subcore_barrier()

        sl = pl.ds(row_base, TILE_ROWS)
        pltpu.sync_copy(deg_sh.at[sl], deg_out.at[c, sl])

    return pl.kernel(
        body,
        out_type=jax.ShapeDtypeStruct((NC, N_PAD, DEGW), jnp.float32),
        mesh=mesh,
        scratch_types=(
            pltpu.VMEM((2, CD), jnp.int32),
            pltpu.VMEM((CD, DEGW), jnp.float32),
            pltpu.VMEM_SHARED((N_PAD, DEGW), jnp.float32),
            pltpu.SemaphoreType.DMA,
            pltpu.SemaphoreType.DMA,
        ))


_sc_edge = _make_sc_kernel()
_sc_deg = _make_deg_kernel()


def kernel(x, edge_index, batch, edge_attr, params):
    p = params
    src = edge_index[0]
    dst = edge_index[1]
    dst_p = jnp.concatenate(
        [dst, jnp.full((E_PAD - E,), N, dtype=jnp.int32)])
    src_p = jnp.concatenate(
        [src, jnp.zeros((E_PAD - E,), dtype=jnp.int32)])

    ea_p = jnp.pad(edge_attr, ((0, E_PAD - E), (0, 0)))
    x_p = jnp.pad(x, ((0, N_PAD - N), (0, 0)))

    zeros = jnp.zeros((TILE_ROWS, H), jnp.float32)
    zeros8 = jnp.zeros((TILE_ROWS, DEGW), jnp.float32)
    ones = jnp.ones((CD, DEGW), jnp.float32)

    h = _mlp_call(x_p, p['emb_W1'], p['emb_b1'], p['emb_W2'], p['emb_b2'],
                  block=1264)

    degt = _sc_deg(dst_p, zeros8, ones)
    for l in range(L):
        W1 = p[f'l{l}_msg_W1']
        A, B = _ab_call(h, W1[:H], W1[H:2 * H], p[f'l{l}_msg_b1'])
        Ce = _ce_call(ea_p, W1[2 * H:])
        Sp = _sc_edge(A, B, Ce, dst_p, src_p, zeros)
        U1 = p[f'l{l}_upd_W1']
        h = _update_call(Sp, degt, h, p[f'l{l}_msg_W2'], p[f'l{l}_msg_b2'],
                         U1[:H], U1[H:], p[f'l{l}_upd_b1'],
                         p[f'l{l}_upd_W2'], p[f'l{l}_upd_b2'])

    out = _mlp_call(h, p['head_W1'], p['head_b1'], p['head_W2'],
                    p['head_b2'], block=1264)
    return out[:N]
